# baseline (speedup 1.0000x reference)
import functools
import math

import jax
import jax.numpy as jnp
from jax import lax
from jax.experimental import pallas as pl
from jax.experimental.pallas import tpu as pltpu
from jax.experimental.pallas import tpu_sc as plsc

E = 8
TOP_K = 2
D = 768
H = 256
EPS_LN = 1e-5
N = 2048

BK = 256
NBLK = 2 * N // BK + E
NSLOT = NBLK * BK
NPAIR = 2 * N
LCAP = NPAIR + BK
LCAP2 = LCAP + 16
BN = 512

NC = 2
NS = 16
NW = NC * NS
GROWS = NSLOT // NW
TOKT = N // NW


def _router_body(x_ref, wr_ref, br_ref, e0_ref, e1_ref, w0_ref, w1_ref):
    xb = x_ref[...]
    logits = jnp.dot(xb, wr_ref[...], preferred_element_type=jnp.float32)
    logits = logits + br_ref[...]
    m = jnp.max(logits, axis=-1, keepdims=True)
    p = jnp.exp(logits - m)
    p = p / jnp.sum(p, axis=-1, keepdims=True)

    iota_e = lax.broadcasted_iota(jnp.int32, (BN, E), 1)
    m1 = jnp.max(p, axis=-1, keepdims=True)
    idx1 = jnp.min(jnp.where(p == m1, iota_e, E), axis=-1, keepdims=True)
    p_rest = jnp.where(iota_e == idx1, -jnp.inf, p)
    m2 = jnp.max(p_rest, axis=-1, keepdims=True)
    idx2 = jnp.min(jnp.where(p_rest == m2, iota_e, E), axis=-1, keepdims=True)
    denom = jnp.maximum(m1 + m2, 1e-9)
    half = 0.5 / denom
    e0_ref[...] = idx1[:, 0]
    e1_ref[...] = idx2[:, 0]
    w0_ref[...] = (m1 * half)[:, 0]
    w1_ref[...] = (m2 * half)[:, 0]


def _router(x2, Wr, br2):
    return pl.pallas_call(
        _router_body,
        grid=(N // BN,),
        in_specs=[
            pl.BlockSpec((BN, D), lambda i: (i, 0)),
            pl.BlockSpec((D, E), lambda i: (0, 0)),
            pl.BlockSpec((1, E), lambda i: (0, 0)),
        ],
        out_specs=[
            pl.BlockSpec((BN,), lambda i: (i,)),
            pl.BlockSpec((BN,), lambda i: (i,)),
            pl.BlockSpec((BN,), lambda i: (i,)),
            pl.BlockSpec((BN,), lambda i: (i,)),
        ],
        out_shape=[
            jax.ShapeDtypeStruct((N,), jnp.int32),
            jax.ShapeDtypeStruct((N,), jnp.int32),
            jax.ShapeDtypeStruct((N,), jnp.float32),
            jax.ShapeDtypeStruct((N,), jnp.float32),
        ],
    )(x2, Wr, br2)


_MESH = plsc.VectorSubcoreMesh(core_axis_name="c", subcore_axis_name="s")


@functools.partial(
    pl.kernel,
    mesh=_MESH,
    compiler_params=pltpu.CompilerParams(needs_layout_passes=False),
    out_type=(
        jax.ShapeDtypeStruct((NSLOT, D), jnp.float32),
        jax.ShapeDtypeStruct((NSLOT,), jnp.float32),
        jax.ShapeDtypeStruct((NPAIR + 512,), jnp.int32),
        jax.ShapeDtypeStruct((32,), jnp.int32),
        jax.ShapeDtypeStruct((NC * NSLOT,), jnp.int32),
    ),
    scratch_types=[
        pltpu.VMEM((NPAIR,), jnp.int32),
        pltpu.VMEM((NPAIR,), jnp.float32),
        pltpu.VMEM((LCAP2,), jnp.int32),
        pltpu.VMEM((LCAP2,), jnp.float32),
        pltpu.VMEM((LCAP2,), jnp.int32),
        pltpu.VMEM((LCAP2,), jnp.int32),
        pltpu.VMEM((NS * 16,), jnp.int32),
        pltpu.VMEM((16,), jnp.int32),
        pltpu.VMEM((32,), jnp.int32),
        pltpu.VMEM((NSLOT // NS,), jnp.int32),
        pltpu.VMEM((NSLOT // NS,), jnp.float32),
        pltpu.VMEM((GROWS // 2,), jnp.int32),
        pltpu.VMEM((GROWS // 2, D), jnp.float32),
        pltpu.VMEM_SHARED((NS * 16,), jnp.int32),
        pltpu.SemaphoreType.DMA,
    ],
)
def _dispatch(e0_hbm, e1_hbm, w0_hbm, w1_hbm, x_hbm,
              xg, swt, inv, map32, rows,
              eidv, wtv, ltok, lwt, lpid, gpos, cnts_all, vtmp, map_v,
              zero_i, zero_f, idxc, grows, cshared, sem):
    cid = lax.axis_index("c")
    sid = lax.axis_index("s")
    g = cid * NS + sid
    iota = lax.iota(jnp.int32, 16)
    zi16 = jnp.zeros((16,), jnp.int32)
    zf16 = jnp.zeros((16,), jnp.float32)

    zchunk = NSLOT // NS

    def _zfill(i, _):
        zero_i[pl.ds(i * 16, 16)] = zi16
        zero_f[pl.ds(i * 16, 16)] = zf16
        return 0

    lax.fori_loop(0, zchunk // 16, _zfill, 0)
    pltpu.sync_copy(zero_i, rows.at[pl.ds(pl.multiple_of(cid * NSLOT + sid * zchunk, 8), zchunk)])
    pltpu.sync_copy(zero_f, swt.at[pl.ds(pl.multiple_of(sid * zchunk, 8), zchunk)])

    @pl.when(sid < E)
    def _():
        pltpu.sync_copy(e0_hbm, eidv.at[pl.ds(0, N)])
        pltpu.sync_copy(e1_hbm, eidv.at[pl.ds(N, N)])
        pltpu.sync_copy(w0_hbm, wtv.at[pl.ds(0, N)])
        pltpu.sync_copy(w1_hbm, wtv.at[pl.ds(N, N)])
        sent = jnp.full((16,), NPAIR, jnp.int32) + sid

        def _pf(i, _):
            ltok[pl.ds(i * 16, 16)] = zi16
            lwt[pl.ds(i * 16, 16)] = zf16
            lpid[pl.ds(i * 16, 16)] = sent
            return 0

        lax.fori_loop(0, LCAP2 // 16, _pf, 0)

        dnums = lax.GatherDimensionNumbers(
            offset_dims=(), collapsed_slice_dims=(0,), start_index_map=(0,))

        def _shift(v, k):
            idx = jnp.maximum(iota - k, 0)
            return lax.gather(v, idx[:, None], dnums, slice_sizes=(1,),
                              mode=lax.GatherScatterMode.PROMISE_IN_BOUNDS)

        def _cbody(j, off):
            ev = eidv[pl.ds(j * 16, 16)]
            wv = wtv[pl.ds(j * 16, 16)]
            pidv = j * 16 + iota
            tokv = pidv & (N - 1)
            mask = ev == sid
            s = jnp.where(mask, 1, 0)
            for k in (1, 2, 4, 8):
                s = s + jnp.where(iota >= k, _shift(s, k), 0)
            posv = jnp.where(mask, off + s - 1, LCAP + iota)
            plsc.store_scatter(ltok, [posv], tokv)
            plsc.store_scatter(lwt, [posv], wv)
            plsc.store_scatter(lpid, [posv], pidv)
            cnt = plsc.all_reduce_population_count(mask)
            return off + jnp.max(cnt)

        c_e = lax.fori_loop(0, NPAIR // 16, _cbody, jnp.int32(0))
        lpid[pl.ds(LCAP, 16)] = sent
        vtmp[...] = jnp.zeros((16,), jnp.int32) + c_e

    @pl.when(sid >= E)
    def _():
        vtmp[...] = zi16

    pltpu.sync_copy(vtmp, cshared.at[pl.ds(pl.multiple_of(sid * 16, 8), 16)])
    plsc.subcore_barrier()
    for t in range(E):
        pltpu.sync_copy(cshared.at[pl.ds(t * 16, 16)],
                        cnts_all.at[pl.ds(t * 16, 16)])

    counts = [jnp.max(cnts_all[pl.ds(t * 16, 16)]) for t in range(E)]
    aligned = [((c + BK - 1) // BK) * BK for c in counts]
    my_base = jnp.int32(0)
    my_aligned = jnp.int32(0)
    for t in range(E):
        my_base = my_base + jnp.where(sid > t, aligned[t], 0)
        my_aligned = my_aligned + jnp.where(sid == t, aligned[t], 0)

    @pl.when(sid < E)
    def _():
        def _cp(jb, _):
            pltpu.sync_copy(
                ltok.at[pl.ds(jb * BK, BK)],
                rows.at[pl.ds(pl.multiple_of(cid * NSLOT + my_base + jb * BK, 8), BK)])
            pltpu.sync_copy(
                lwt.at[pl.ds(jb * BK, BK)],
                swt.at[pl.ds(pl.multiple_of(my_base + jb * BK, 8), BK)])
            return 0

        lax.fori_loop(0, my_aligned // BK, _cp, 0)

        def _gp(i, _):
            gpos[pl.ds(i * 16, 16)] = my_base + i * 16 + iota
            return 0

        lax.fori_loop(0, LCAP2 // 16, _gp, 0)
        pltpu.async_copy(gpos, inv.at[lpid], sem).wait()

    @pl.when(sid == E)
    def _():
        m0 = jnp.zeros((16,), jnp.int32)
        m1 = jnp.zeros((16,), jnp.int32)
        cumb = jnp.int32(0)
        for t in range(E):
            cumb = cumb + aligned[t] // BK
            m0 = m0 + jnp.where(iota >= cumb, 1, 0)
            m1 = m1 + jnp.where(iota + 16 >= cumb, 1, 0)
        map_v[pl.ds(0, 16)] = jnp.minimum(m0, E - 1)
        map_v[pl.ds(16, 16)] = jnp.minimum(m1, E - 1)
        pltpu.sync_copy(map_v, map32)

    plsc.subcore_barrier()

    base_row = g * GROWS
    for c in range(2):
        pltpu.sync_copy(
            rows.at[pl.ds(pl.multiple_of(cid * NSLOT + base_row + c * (GROWS // 2), 8),
                          GROWS // 2)], idxc)
        pltpu.async_copy(x_hbm.at[idxc], grows, sem).wait()
        pltpu.sync_copy(
            grows, xg.at[pl.ds(pl.multiple_of(base_row + c * (GROWS // 2), 8), GROWS // 2)])


def _group_body(map_ref, xg_ref, w1_ref, b1_ref, g1_ref, beta1_ref,
                w2_ref, b2_ref, swt_ref, out_ref):
    del map_ref
    xb = xg_ref[...]
    h = jnp.dot(xb, w1_ref[0], preferred_element_type=jnp.float32)
    h = h + b1_ref[0]
    s1 = jnp.sum(h, axis=-1, keepdims=True)
    s2 = jnp.sum(h * h, axis=-1, keepdims=True)
    mu = s1 * (1.0 / H)
    var = s2 * (1.0 / H) - mu * mu
    hn = (h - mu) * lax.rsqrt(var + EPS_LN)
    hn = hn * g1_ref[0] + beta1_ref[0]
    swc = jnp.transpose(swt_ref[0], (1, 0))
    t = swc * hn
    a = t + t * lax.erf(hn * (1.0 / math.sqrt(2.0)))
    out = jnp.dot(a, w2_ref[0], preferred_element_type=jnp.float32)
    out_ref[...] = out + (2.0 * swc) * b2_ref[0]


def _grouped(map32, xg, W1, b1, g1, beta1, W2, b2, swt3):
    grid_spec = pltpu.PrefetchScalarGridSpec(
        num_scalar_prefetch=1,
        grid=(NBLK,),
        in_specs=[
            pl.BlockSpec((BK, D), lambda i, m: (i, 0)),
            pl.BlockSpec((1, D, H), lambda i, m: (m[i], 0, 0)),
            pl.BlockSpec((1, 1, H), lambda i, m: (m[i], 0, 0)),
            pl.BlockSpec((1, 1, H), lambda i, m: (m[i], 0, 0)),
            pl.BlockSpec((1, 1, H), lambda i, m: (m[i], 0, 0)),
            pl.BlockSpec((1, H, D), lambda i, m: (m[i], 0, 0)),
            pl.BlockSpec((1, 1, D), lambda i, m: (m[i], 0, 0)),
            pl.BlockSpec((1, 1, BK), lambda i, m: (i, 0, 0)),
        ],
        out_specs=pl.BlockSpec((BK, D), lambda i, m: (i, 0)),
    )
    return pl.pallas_call(
        _group_body,
        grid_spec=grid_spec,
        out_shape=jax.ShapeDtypeStruct((NSLOT, D), jnp.float32),
    )(map32, xg, W1, b1, g1, beta1, W2, b2, swt3)


@functools.partial(
    pl.kernel,
    mesh=_MESH,
    compiler_params=pltpu.CompilerParams(needs_layout_passes=False),
    out_type=jax.ShapeDtypeStruct((N, D), jnp.float32),
    scratch_types=[
        pltpu.VMEM((TOKT,), jnp.int32),
        pltpu.VMEM((TOKT,), jnp.int32),
        pltpu.VMEM((TOKT, D), jnp.float32),
        pltpu.VMEM((TOKT, D), jnp.float32),
        pltpu.SemaphoreType.DMA,
    ],
)
def _combine(out_hbm, inv_hbm, y_hbm, i0, i1, r0, r1, sem):
    cid = lax.axis_index("c")
    sid = lax.axis_index("s")
    g = cid * NS + sid
    base = g * TOKT
    pltpu.sync_copy(inv_hbm.at[pl.ds(pl.multiple_of(base, 8), TOKT)], i0)
    pltpu.sync_copy(inv_hbm.at[pl.ds(pl.multiple_of(N + base, 8), TOKT)], i1)
    pltpu.async_copy(out_hbm.at[i0], r0, sem).wait()
    pltpu.async_copy(out_hbm.at[i1], r1, sem).wait()

    def _row(i, _):
        for j in range(D // 16):
            sl = pl.ds(j * 16, 16)
            r0[i, sl] = r0[i, sl] + r1[i, sl]
        return 0

    lax.fori_loop(0, TOKT, _row, 0)
    pltpu.sync_copy(r0, y_hbm.at[pl.ds(pl.multiple_of(base, 8), TOKT)])


@jax.jit
def kernel(x, Wr, br, W1, b1, g1, beta1, W2, b2):
    orig_shape = x.shape
    x2 = x.reshape(N, D)
    e0, e1, w0, w1 = _router(x2, Wr, br.reshape(1, E))
    xg, swt, inv, map32, _ = _dispatch(e0, e1, w0, w1, x2)
    out = _grouped(map32, xg, W1, b1.reshape(E, 1, H), g1.reshape(E, 1, H),
                   beta1.reshape(E, 1, H), W2, b2.reshape(E, 1, D),
                   swt.reshape(NBLK, 1, BK))
    y = _combine(out, inv)
    return y.reshape(orig_shape)


# speedup vs baseline: 256.7797x; 256.7797x over previous
import functools
import math

import jax
import jax.numpy as jnp
from jax import lax
from jax.experimental import pallas as pl

E = 8
TOP_K = 2
D = 768
H = 256
EPS_LN = 1e-5

BN = 512


def _moe_body(x_ref, wr_ref, br_ref, w1_ref, b1_ref, g1_ref, beta1_ref,
              w2cat_ref, b2_ref, out_ref):
    xb = x_ref[...]

    logits = jnp.dot(xb, wr_ref[...], preferred_element_type=jnp.float32)
    logits = logits + br_ref[...]
    m = jnp.max(logits, axis=-1, keepdims=True)
    p = jnp.exp(logits - m)
    p = p / jnp.sum(p, axis=-1, keepdims=True)

    iota_e = lax.broadcasted_iota(jnp.int32, (BN, E), 1)
    m1 = jnp.max(p, axis=-1, keepdims=True)
    idx1 = jnp.min(jnp.where(p == m1, iota_e, E), axis=-1, keepdims=True)
    mask1 = iota_e == idx1
    p_rest = jnp.where(mask1, -jnp.inf, p)
    m2 = jnp.max(p_rest, axis=-1, keepdims=True)
    idx2 = jnp.min(jnp.where(p_rest == m2, iota_e, E), axis=-1, keepdims=True)
    mask2 = iota_e == idx2
    denom = jnp.maximum(m1 + m2, 1e-9)
    w = jnp.where(mask1 | mask2, p, 0.0) / denom

    b1 = b1_ref[...]
    g1 = g1_ref[...]
    beta1 = beta1_ref[...]

    inv_sqrt2 = 1.0 / math.sqrt(2.0)
    w_half = 0.5 * w
    chunks = []
    for e in range(E):
        h = jnp.dot(xb, w1_ref[e], preferred_element_type=jnp.float32)
        h = h + b1[e][None, :]
        s1 = jnp.sum(h, axis=-1, keepdims=True)
        s2 = jnp.sum(h * h, axis=-1, keepdims=True)
        mu = s1 * (1.0 / H)
        var = s2 * (1.0 / H) - mu * mu
        hn = (h - mu) * lax.rsqrt(var + EPS_LN)
        hn = hn * g1[e][None, :] + beta1[e][None, :]
        t = w_half[:, e][:, None] * hn
        chunks.append(t + t * lax.erf(hn * inv_sqrt2))
    a_all = jnp.concatenate(chunks, axis=-1)
    acc = jnp.dot(a_all, w2cat_ref[...], preferred_element_type=jnp.float32)
    acc = acc + jnp.dot(w, b2_ref[...], preferred_element_type=jnp.float32)
    out_ref[...] = acc


@functools.partial(jax.jit, static_argnames=("interpret",))
def kernel(x, Wr, br, W1, b1, g1, beta1, W2, b2, interpret=False):
    orig_shape = x.shape
    x2 = x.reshape(-1, x.shape[-1])
    n = x2.shape[0]
    grid = (n // BN,)
    out = pl.pallas_call(
        _moe_body,
        grid=grid,
        in_specs=[
            pl.BlockSpec((BN, D), lambda i: (i, 0)),
            pl.BlockSpec((D, E), lambda i: (0, 0)),
            pl.BlockSpec((1, E), lambda i: (0, 0)),
            pl.BlockSpec((E, D, H), lambda i: (0, 0, 0)),
            pl.BlockSpec((E, H), lambda i: (0, 0)),
            pl.BlockSpec((E, H), lambda i: (0, 0)),
            pl.BlockSpec((E, H), lambda i: (0, 0)),
            pl.BlockSpec((E * H, D), lambda i: (0, 0)),
            pl.BlockSpec((E, D), lambda i: (0, 0)),
        ],
        out_specs=pl.BlockSpec((BN, D), lambda i: (i, 0)),
        out_shape=jax.ShapeDtypeStruct((n, D), jnp.float32),
        interpret=interpret,
    )(x2, Wr, br.reshape(1, E), W1, b1, g1, beta1,
      W2.reshape(E * H, D), b2)
    return out.reshape(orig_shape)


# speedup vs baseline: 257.0010x; 1.0009x over previous
import functools
import math

import jax
import jax.numpy as jnp
from jax import lax
from jax.experimental import pallas as pl

E = 8
TOP_K = 2
D = 768
H = 256
EPS_LN = 1e-5

BN = 512


def _moe_body(x_ref, wr_ref, br_ref, w1_ref, b1_ref, g1_ref, beta1_ref,
              w2cat_ref, b2_ref, out_ref):
    xb = x_ref[...]

    logits = jnp.dot(xb, wr_ref[...], preferred_element_type=jnp.float32)
    logits = logits + br_ref[...]
    m = jnp.max(logits, axis=-1, keepdims=True)
    p = jnp.exp(logits - m)
    p = p / jnp.sum(p, axis=-1, keepdims=True)

    iota_e = lax.broadcasted_iota(jnp.int32, (BN, E), 1)
    m1 = jnp.max(p, axis=-1, keepdims=True)
    idx1 = jnp.min(jnp.where(p == m1, iota_e, E), axis=-1, keepdims=True)
    mask1 = iota_e == idx1
    p_rest = jnp.where(mask1, -jnp.inf, p)
    m2 = jnp.max(p_rest, axis=-1, keepdims=True)
    idx2 = jnp.min(jnp.where(p_rest == m2, iota_e, E), axis=-1, keepdims=True)
    mask2 = iota_e == idx2
    denom = jnp.maximum(m1 + m2, 1e-9)
    w = jnp.where(mask1 | mask2, p, 0.0) / denom

    b1 = b1_ref[...]
    g1 = g1_ref[...]
    beta1 = beta1_ref[...]

    inv_sqrt2 = 1.0 / math.sqrt(2.0)
    w_half = 0.5 * w
    chunks = []
    for e in range(E):
        h = jnp.dot(xb, w1_ref[e], preferred_element_type=jnp.float32)
        h = h + b1[e][None, :]
        s1 = jnp.sum(h, axis=-1, keepdims=True)
        s2 = jnp.sum(h * h, axis=-1, keepdims=True)
        mu = s1 * (1.0 / H)
        var = s2 * (1.0 / H) - mu * mu
        hn = (h - mu) * lax.rsqrt(var + EPS_LN)
        hn = hn * g1[e][None, :] + beta1[e][None, :]
        t = w_half[:, e][:, None] * hn
        chunks.append(t + t * lax.erf(hn * inv_sqrt2))
    a_all = jnp.concatenate(chunks, axis=-1)
    acc = jnp.dot(a_all, w2cat_ref[...], preferred_element_type=jnp.float32)
    acc = acc + jnp.dot(w, b2_ref[...], preferred_element_type=jnp.float32)
    out_ref[...] = acc


@jax.jit
def kernel(x, Wr, br, W1, b1, g1, beta1, W2, b2):
    orig_shape = x.shape
    x2 = x.reshape(-1, x.shape[-1])
    n = x2.shape[0]
    grid = (n // BN,)
    out = pl.pallas_call(
        _moe_body,
        grid=grid,
        in_specs=[
            pl.BlockSpec((BN, D), lambda i: (i, 0)),
            pl.BlockSpec((D, E), lambda i: (0, 0)),
            pl.BlockSpec((1, E), lambda i: (0, 0)),
            pl.BlockSpec((E, D, H), lambda i: (0, 0, 0)),
            pl.BlockSpec((E, H), lambda i: (0, 0)),
            pl.BlockSpec((E, H), lambda i: (0, 0)),
            pl.BlockSpec((E, H), lambda i: (0, 0)),
            pl.BlockSpec((E * H, D), lambda i: (0, 0)),
            pl.BlockSpec((E, D), lambda i: (0, 0)),
        ],
        out_specs=pl.BlockSpec((BN, D), lambda i: (i, 0)),
        out_shape=jax.ShapeDtypeStruct((n, D), jnp.float32),
    )(x2, Wr, br.reshape(1, E), W1, b1, g1, beta1,
      W2.reshape(E * H, D), b2)
    return out.reshape(orig_shape)
